# baseline (speedup 1.0000x reference)
import functools

import jax
import jax.numpy as jnp
from jax import lax
from jax.experimental import pallas as pl
from jax.experimental.pallas import tpu as pltpu
from jax.experimental.pallas import tpu_sc as plsc

VOCAB = 1000000
EMBED_DIM = 32
HIDDEN_DIM = 64
NUM_TIERS = 4
BATCH = 16384
HIST_LEN = 200

_L = 16
_CW = 1024
_NCHUNK = VOCAB // _CW
_TAIL = VOCAB - _NCHUNK * _CW


def _make_lin_kernel():
    info = plsc.get_sparse_core_info()
    nc, ns = info.num_cores, info.num_subcores
    nw = nc * ns
    iters = (_NCHUNK + 1 + nw - 1) // nw

    mesh = plsc.VectorSubcoreMesh(core_axis_name="c", subcore_axis_name="s")

    @functools.partial(
        pl.kernel,
        mesh=mesh,
        out_type=jax.ShapeDtypeStruct((VOCAB, EMBED_DIM), jnp.float32),
        scratch_types=[
            pltpu.VMEM((EMBED_DIM, _CW), jnp.float32),
            pltpu.VMEM((_CW, EMBED_DIM), jnp.float32),
        ],
        compiler_params=pltpu.CompilerParams(
            use_tc_tiling_on_sc=False, needs_layout_passes=False
        ),
    )
    def lin(tt_hbm, out_hbm, inb, outb, *, _iters=iters):
        wid = lax.axis_index("s") * nc + lax.axis_index("c")
        rows0 = lax.iota(jnp.int32, _L)
        rows1 = rows0 + _L

        def do_chunk(base, n):
            pltpu.sync_copy(tt_hbm.at[:, pl.ds(base, n)], inb.at[:, pl.ds(0, n)])

            def tr_body(j, carry):
                r = j * 4
                for u in range(4):
                    col = jnp.full((_L,), r + u, jnp.int32)
                    a = plsc.load_gather(inb, [rows0, col])
                    b = plsc.load_gather(inb, [rows1, col])
                    outb[r + u, pl.ds(0, _L)] = a
                    outb[r + u, pl.ds(_L, _L)] = b
                return carry

            lax.fori_loop(0, n // 4, tr_body, 0)
            pltpu.sync_copy(outb.at[pl.ds(0, n)], out_hbm.at[pl.ds(base, n)])

        def it_body(it, carry):
            chunk = wid + it * nw

            @pl.when(chunk < _NCHUNK)
            def _():
                do_chunk(chunk * _CW, _CW)

            @pl.when(chunk == _NCHUNK)
            def _():
                do_chunk(_NCHUNK * _CW, _TAIL)

            return carry

        lax.fori_loop(0, _iters, it_body, 0)

    return lin


_lin = _make_lin_kernel()


def _make_pool_kernel():
    info = plsc.get_sparse_core_info()
    nc, ns = info.num_cores, info.num_subcores
    nw = nc * ns
    S = BATCH // nw
    CH = 128
    NCH = S // CH
    NBUF = 4

    mesh = plsc.VectorSubcoreMesh(core_axis_name="c", subcore_axis_name="s")

    @functools.partial(
        pl.kernel,
        mesh=mesh,
        out_type=jax.ShapeDtypeStruct((BATCH, EMBED_DIM), jnp.float32),
        scratch_types=[
            pltpu.VMEM((CH, HIST_LEN), jnp.int32),
            pltpu.VMEM((NBUF, HIST_LEN, EMBED_DIM), jnp.float32),
            pltpu.VMEM((CH, EMBED_DIM), jnp.float32),
            pltpu.SemaphoreType.DMA,
            pltpu.SemaphoreType.DMA,
            pltpu.SemaphoreType.DMA,
            pltpu.SemaphoreType.DMA,
        ],
        compiler_params=pltpu.CompilerParams(use_tc_tiling_on_sc=False),
    )
    def pool(ids_hbm, table_hbm, out_hbm, ids_v, rows_v, sums_v, *sems):
        wid = lax.axis_index("s") * nc + lax.axis_index("c")
        base = wid * S

        def start_gather(i, b):
            pltpu.async_copy(
                table_hbm.at[ids_v.at[i, pl.ds(0, 128)]],
                rows_v.at[b, pl.ds(0, 128)],
                sems[b],
            )
            pltpu.async_copy(
                table_hbm.at[ids_v.at[i, pl.ds(128, HIST_LEN - 128)]],
                rows_v.at[b, pl.ds(128, HIST_LEN - 128)],
                sems[b],
            )

        def wait_gather(b):
            pltpu.make_async_copy(
                table_hbm.at[pl.ds(0, HIST_LEN)], rows_v.at[b], sems[b]
            ).wait()

        def accum_store(i, b):
            def acc_body(j, accs):
                accs = list(accs)
                r = j * 8
                for u in range(8):
                    accs[u % 4] = accs[u % 4] + rows_v[b, r + u, pl.ds(0, _L)]
                    accs[4 + u % 4] = accs[4 + u % 4] + rows_v[b, r + u, pl.ds(_L, _L)]
                return tuple(accs)

            z = jnp.zeros((_L,), jnp.float32)
            accs = lax.fori_loop(0, HIST_LEN // 8, acc_body, (z,) * 8)
            sums_v[i, pl.ds(0, _L)] = (accs[0] + accs[1]) + (accs[2] + accs[3])
            sums_v[i, pl.ds(_L, _L)] = (accs[4] + accs[5]) + (accs[6] + accs[7])

        def chunk_body(c, carry):
            cbase = base + c * CH
            pltpu.sync_copy(ids_hbm.at[pl.ds(cbase, CH)], ids_v)
            for b in range(NBUF - 1):
                start_gather(b, b)

            def group_body(g, carry2):
                for b in range(NBUF):
                    i = g * NBUF + b

                    @pl.when(i + NBUF - 1 < CH)
                    def _():
                        start_gather(i + NBUF - 1, (b + NBUF - 1) % NBUF)

                    wait_gather(b)
                    accum_store(i, b)
                return carry2

            lax.fori_loop(0, CH // NBUF, group_body, 0)
            pltpu.sync_copy(sums_v, out_hbm.at[pl.ds(cbase, CH)])
            return carry

        lax.fori_loop(0, NCH, chunk_body, 0)

    return pool


_pool = _make_pool_kernel()


def _mlp_body(ids_ref, sums_ref, w1_ref, b1_ref, w2_ref, b2_ref, out_ref):
    ids = ids_ref[...]
    valid = jnp.sum((ids != 0).astype(jnp.float32), axis=1, keepdims=True)
    rep = sums_ref[...] / jnp.maximum(valid, 1.0)
    x = jnp.dot(rep, w1_ref[...], preferred_element_type=jnp.float32) + b1_ref[...]
    x = jnp.maximum(x, 0.0)
    out_ref[...] = (
        jnp.dot(x, w2_ref[...], preferred_element_type=jnp.float32) + b2_ref[...]
    )


def _mlp(ids, sums, w1t, b1r, w2t, b2r):
    BT = 1024
    return pl.pallas_call(
        _mlp_body,
        grid=(BATCH // BT,),
        in_specs=[
            pl.BlockSpec((BT, HIST_LEN), lambda i: (i, 0)),
            pl.BlockSpec((BT, EMBED_DIM), lambda i: (i, 0)),
            pl.BlockSpec((EMBED_DIM, HIDDEN_DIM), lambda i: (0, 0)),
            pl.BlockSpec((1, HIDDEN_DIM), lambda i: (0, 0)),
            pl.BlockSpec((HIDDEN_DIM, NUM_TIERS), lambda i: (0, 0)),
            pl.BlockSpec((1, NUM_TIERS), lambda i: (0, 0)),
        ],
        out_specs=pl.BlockSpec((BT, NUM_TIERS), lambda i: (i, 0)),
        out_shape=jax.ShapeDtypeStruct((BATCH, NUM_TIERS), jnp.float32),
    )(ids, sums, w1t, b1r, w2t, b2r)


def kernel(input_ids, table, W1, b1, W2, b2):
    lin = _lin(table.T)
    sums = _pool(input_ids, lin)
    return _mlp(
        input_ids,
        sums,
        W1.T,
        b1.reshape(1, HIDDEN_DIM),
        W2.T,
        b2.reshape(1, NUM_TIERS),
    )


# speedup vs baseline: 7.0647x; 7.0647x over previous
import functools

import jax
import jax.numpy as jnp
from jax import lax
from jax.experimental import pallas as pl
from jax.experimental.pallas import tpu as pltpu
from jax.experimental.pallas import tpu_sc as plsc

VOCAB = 1000000
EMBED_DIM = 32
HIDDEN_DIM = 64
NUM_TIERS = 4
BATCH = 16384
HIST_LEN = 200

_L = 16


def _make_pool_kernel():
    info = plsc.get_sparse_core_info()
    nc, ns = info.num_cores, info.num_subcores
    nw = nc * ns
    S = BATCH // nw
    CH = 128
    NCH = S // CH
    NBUF = 8

    mesh = plsc.VectorSubcoreMesh(core_axis_name="c", subcore_axis_name="s")

    @functools.partial(
        pl.kernel,
        mesh=mesh,
        out_type=jax.ShapeDtypeStruct((BATCH, EMBED_DIM), jnp.float32),
        scratch_types=[
            pltpu.VMEM((CH, HIST_LEN), jnp.int32),
            pltpu.VMEM((NBUF, HIST_LEN, EMBED_DIM), jnp.float32),
            pltpu.VMEM((CH, EMBED_DIM), jnp.float32),
            pltpu.SemaphoreType.DMA,
            pltpu.SemaphoreType.DMA,
            pltpu.SemaphoreType.DMA,
            pltpu.SemaphoreType.DMA,
            pltpu.SemaphoreType.DMA,
            pltpu.SemaphoreType.DMA,
            pltpu.SemaphoreType.DMA,
            pltpu.SemaphoreType.DMA,
        ],
        compiler_params=pltpu.CompilerParams(use_tc_tiling_on_sc=False),
    )
    def pool(ids_hbm, table_hbm, out_hbm, ids_v, rows_v, sums_v, *sems):
        wid = lax.axis_index("s") * nc + lax.axis_index("c")
        base = wid * S

        def start_gather(i, b):
            pltpu.async_copy(
                table_hbm.at[ids_v.at[i, pl.ds(0, 128)]],
                rows_v.at[b, pl.ds(0, 128)],
                sems[b],
            )
            pltpu.async_copy(
                table_hbm.at[ids_v.at[i, pl.ds(128, HIST_LEN - 128)]],
                rows_v.at[b, pl.ds(128, HIST_LEN - 128)],
                sems[b],
            )

        def wait_gather(b):
            pltpu.make_async_copy(
                table_hbm.at[pl.ds(0, HIST_LEN)], rows_v.at[b], sems[b]
            ).wait()

        def accum_store(i, b):
            def acc_body(j, accs):
                accs = list(accs)
                r = j * 8
                for u in range(8):
                    accs[u % 4] = accs[u % 4] + rows_v[b, r + u, pl.ds(0, _L)]
                    accs[4 + u % 4] = accs[4 + u % 4] + rows_v[b, r + u, pl.ds(_L, _L)]
                return tuple(accs)

            z = jnp.zeros((_L,), jnp.float32)
            accs = lax.fori_loop(0, HIST_LEN // 8, acc_body, (z,) * 8)
            sums_v[i, pl.ds(0, _L)] = (accs[0] + accs[1]) + (accs[2] + accs[3])
            sums_v[i, pl.ds(_L, _L)] = (accs[4] + accs[5]) + (accs[6] + accs[7])

        def chunk_body(c, carry):
            cbase = base + c * CH
            pltpu.sync_copy(ids_hbm.at[pl.ds(cbase, CH)], ids_v)
            for b in range(NBUF - 1):
                start_gather(b, b)

            def group_body(g, carry2):
                for b in range(NBUF):
                    i = g * NBUF + b

                    @pl.when(i + NBUF - 1 < CH)
                    def _():
                        start_gather(i + NBUF - 1, (b + NBUF - 1) % NBUF)

                    wait_gather(b)
                    accum_store(i, b)
                return carry2

            lax.fori_loop(0, CH // NBUF, group_body, 0)
            pltpu.sync_copy(sums_v, out_hbm.at[pl.ds(cbase, CH)])
            return carry

        lax.fori_loop(0, NCH, chunk_body, 0)

    return pool


_pool = _make_pool_kernel()


_QP = 1 << 18


def _lin_body(t0, t1, t2, t3, out_ref):
    out_ref[...] = jnp.concatenate(
        [t0[...].T, t1[...].T, t2[...].T, t3[...].T], axis=1
    )


def _linearize(table_t):
    BM = 8192
    G = _QP // BM
    last_blk = VOCAB // BM
    return pl.pallas_call(
        _lin_body,
        grid=(G,),
        in_specs=[
            pl.BlockSpec(
                (EMBED_DIM, BM),
                lambda g, k=k: (0, jnp.minimum(k * G + g, last_blk)),
            )
            for k in range(4)
        ],
        out_specs=pl.BlockSpec((BM, 128), lambda g: (g, 0)),
        out_shape=jax.ShapeDtypeStruct((_QP, 128), jnp.float32),
        compiler_params=pltpu.CompilerParams(fuse_transposed_lhs_in_matmul=True),
    )(table_t, table_t, table_t, table_t)


def _remap_body(ids_ref, out_ref):
    v = ids_ref[...]
    out_ref[...] = ((v & (_QP - 1)) << 2) | (v >> 18)


def _remap(ids):
    BT = 1024
    return pl.pallas_call(
        _remap_body,
        grid=(BATCH // BT,),
        in_specs=[pl.BlockSpec((BT, HIST_LEN), lambda i: (i, 0))],
        out_specs=pl.BlockSpec((BT, HIST_LEN), lambda i: (i, 0)),
        out_shape=jax.ShapeDtypeStruct((BATCH, HIST_LEN), jnp.int32),
    )(ids)


def _mlp_body(ids_ref, sums_ref, w1_ref, b1_ref, w2_ref, b2_ref, out_ref):
    ids = ids_ref[...]
    valid = jnp.sum((ids != 0).astype(jnp.float32), axis=1, keepdims=True)
    rep = sums_ref[...] / jnp.maximum(valid, 1.0)
    x = jnp.dot(rep, w1_ref[...], preferred_element_type=jnp.float32) + b1_ref[...]
    x = jnp.maximum(x, 0.0)
    out_ref[...] = (
        jnp.dot(x, w2_ref[...], preferred_element_type=jnp.float32) + b2_ref[...]
    )


def _mlp(ids, sums, w1t, b1r, w2t, b2r):
    BT = 1024
    return pl.pallas_call(
        _mlp_body,
        grid=(BATCH // BT,),
        in_specs=[
            pl.BlockSpec((BT, HIST_LEN), lambda i: (i, 0)),
            pl.BlockSpec((BT, EMBED_DIM), lambda i: (i, 0)),
            pl.BlockSpec((EMBED_DIM, HIDDEN_DIM), lambda i: (0, 0)),
            pl.BlockSpec((1, HIDDEN_DIM), lambda i: (0, 0)),
            pl.BlockSpec((HIDDEN_DIM, NUM_TIERS), lambda i: (0, 0)),
            pl.BlockSpec((1, NUM_TIERS), lambda i: (0, 0)),
        ],
        out_specs=pl.BlockSpec((BT, NUM_TIERS), lambda i: (i, 0)),
        out_shape=jax.ShapeDtypeStruct((BATCH, NUM_TIERS), jnp.float32),
    )(ids, sums, w1t, b1r, w2t, b2r)


def kernel(input_ids, table, W1, b1, W2, b2):
    lin = _linearize(table.T).reshape(4 * _QP, EMBED_DIM)
    rids = _remap(input_ids)
    sums = _pool(rids, lin)
    return _mlp(
        input_ids,
        sums,
        W1.T,
        b1.reshape(1, HIDDEN_DIM),
        W2.T,
        b2.reshape(1, NUM_TIERS),
    )


# speedup vs baseline: 7.1323x; 1.0096x over previous
import functools

import jax
import jax.numpy as jnp
from jax import lax
from jax.experimental import pallas as pl
from jax.experimental.pallas import tpu as pltpu
from jax.experimental.pallas import tpu_sc as plsc

VOCAB = 1000000
EMBED_DIM = 32
HIDDEN_DIM = 64
NUM_TIERS = 4
BATCH = 16384
HIST_LEN = 200

_L = 16


def _make_pool_kernel():
    info = plsc.get_sparse_core_info()
    nc, ns = info.num_cores, info.num_subcores
    nw = nc * ns
    S = BATCH // nw
    CH = 256
    NCH = S // CH
    NBUF = 8

    mesh = plsc.VectorSubcoreMesh(core_axis_name="c", subcore_axis_name="s")

    @functools.partial(
        pl.kernel,
        mesh=mesh,
        out_type=jax.ShapeDtypeStruct((BATCH, EMBED_DIM), jnp.float32),
        scratch_types=[
            pltpu.VMEM((CH, HIST_LEN), jnp.int32),
            pltpu.VMEM((NBUF, HIST_LEN, EMBED_DIM), jnp.float32),
            pltpu.VMEM((CH, EMBED_DIM), jnp.float32),
            pltpu.SemaphoreType.DMA,
            pltpu.SemaphoreType.DMA,
            pltpu.SemaphoreType.DMA,
            pltpu.SemaphoreType.DMA,
            pltpu.SemaphoreType.DMA,
            pltpu.SemaphoreType.DMA,
            pltpu.SemaphoreType.DMA,
            pltpu.SemaphoreType.DMA,
        ],
        compiler_params=pltpu.CompilerParams(use_tc_tiling_on_sc=False),
    )
    def pool(ids_hbm, table_hbm, out_hbm, ids_v, rows_v, sums_v, *sems):
        wid = lax.axis_index("s") * nc + lax.axis_index("c")
        base = wid * S

        def start_gather(i, b):
            pltpu.async_copy(
                table_hbm.at[ids_v.at[i, pl.ds(0, 128)]],
                rows_v.at[b, pl.ds(0, 128)],
                sems[b],
            )
            pltpu.async_copy(
                table_hbm.at[ids_v.at[i, pl.ds(128, HIST_LEN - 128)]],
                rows_v.at[b, pl.ds(128, HIST_LEN - 128)],
                sems[b],
            )

        def wait_gather(b):
            pltpu.make_async_copy(
                table_hbm.at[pl.ds(0, HIST_LEN)], rows_v.at[b], sems[b]
            ).wait()

        def accum_store(i, b):
            def acc_body(j, accs):
                accs = list(accs)
                r = j * 8
                for u in range(8):
                    accs[u % 4] = accs[u % 4] + rows_v[b, r + u, pl.ds(0, _L)]
                    accs[4 + u % 4] = accs[4 + u % 4] + rows_v[b, r + u, pl.ds(_L, _L)]
                return tuple(accs)

            z = jnp.zeros((_L,), jnp.float32)
            accs = lax.fori_loop(0, HIST_LEN // 8, acc_body, (z,) * 8)
            sums_v[i, pl.ds(0, _L)] = (accs[0] + accs[1]) + (accs[2] + accs[3])
            sums_v[i, pl.ds(_L, _L)] = (accs[4] + accs[5]) + (accs[6] + accs[7])

        def chunk_body(c, carry):
            cbase = base + c * CH
            pltpu.sync_copy(ids_hbm.at[pl.ds(cbase, CH)], ids_v)
            for b in range(NBUF - 1):
                start_gather(b, b)

            def group_body(g, carry2):
                for b in range(NBUF):
                    i = g * NBUF + b

                    @pl.when(i + NBUF - 1 < CH)
                    def _():
                        start_gather(i + NBUF - 1, (b + NBUF - 1) % NBUF)

                    wait_gather(b)
                    accum_store(i, b)
                return carry2

            lax.fori_loop(0, CH // NBUF, group_body, 0)
            pltpu.sync_copy(sums_v, out_hbm.at[pl.ds(cbase, CH)])
            return carry

        lax.fori_loop(0, NCH, chunk_body, 0)

    return pool


_pool = _make_pool_kernel()


_QP = 1 << 18


def _lin_body(t0, t1, t2, t3, out_ref):
    out_ref[...] = jnp.concatenate(
        [t0[...].T, t1[...].T, t2[...].T, t3[...].T], axis=1
    )


def _linearize(table_t):
    BM = 8192
    G = _QP // BM
    last_blk = VOCAB // BM
    return pl.pallas_call(
        _lin_body,
        grid=(G,),
        in_specs=[
            pl.BlockSpec(
                (EMBED_DIM, BM),
                lambda g, k=k: (0, jnp.minimum(k * G + g, last_blk)),
            )
            for k in range(4)
        ],
        out_specs=pl.BlockSpec((BM, 128), lambda g: (g, 0)),
        out_shape=jax.ShapeDtypeStruct((_QP, 128), jnp.float32),
        compiler_params=pltpu.CompilerParams(fuse_transposed_lhs_in_matmul=True),
    )(table_t, table_t, table_t, table_t)


def _remap_body(ids_ref, out_ref):
    v = ids_ref[...]
    out_ref[...] = ((v & (_QP - 1)) << 2) | (v >> 18)


def _remap(ids):
    BT = 1024
    return pl.pallas_call(
        _remap_body,
        grid=(BATCH // BT,),
        in_specs=[pl.BlockSpec((BT, HIST_LEN), lambda i: (i, 0))],
        out_specs=pl.BlockSpec((BT, HIST_LEN), lambda i: (i, 0)),
        out_shape=jax.ShapeDtypeStruct((BATCH, HIST_LEN), jnp.int32),
    )(ids)


def _mlp_body(ids_ref, sums_ref, w1_ref, b1_ref, w2_ref, b2_ref, out_ref):
    ids = ids_ref[...]
    valid = jnp.sum((ids != 0).astype(jnp.float32), axis=1, keepdims=True)
    rep = sums_ref[...] / jnp.maximum(valid, 1.0)
    x = jnp.dot(rep, w1_ref[...], preferred_element_type=jnp.float32) + b1_ref[...]
    x = jnp.maximum(x, 0.0)
    out_ref[...] = (
        jnp.dot(x, w2_ref[...], preferred_element_type=jnp.float32) + b2_ref[...]
    )


def _mlp(ids, sums, w1t, b1r, w2t, b2r):
    BT = 1024
    return pl.pallas_call(
        _mlp_body,
        grid=(BATCH // BT,),
        in_specs=[
            pl.BlockSpec((BT, HIST_LEN), lambda i: (i, 0)),
            pl.BlockSpec((BT, EMBED_DIM), lambda i: (i, 0)),
            pl.BlockSpec((EMBED_DIM, HIDDEN_DIM), lambda i: (0, 0)),
            pl.BlockSpec((1, HIDDEN_DIM), lambda i: (0, 0)),
            pl.BlockSpec((HIDDEN_DIM, NUM_TIERS), lambda i: (0, 0)),
            pl.BlockSpec((1, NUM_TIERS), lambda i: (0, 0)),
        ],
        out_specs=pl.BlockSpec((BT, NUM_TIERS), lambda i: (i, 0)),
        out_shape=jax.ShapeDtypeStruct((BATCH, NUM_TIERS), jnp.float32),
    )(ids, sums, w1t, b1r, w2t, b2r)


def kernel(input_ids, table, W1, b1, W2, b2):
    lin = _linearize(table.T).reshape(4 * _QP, EMBED_DIM)
    rids = _remap(input_ids)
    sums = _pool(rids, lin)
    return _mlp(
        input_ids,
        sums,
        W1.T,
        b1.reshape(1, HIDDEN_DIM),
        W2.T,
        b2.reshape(1, NUM_TIERS),
    )


# speedup vs baseline: 7.1385x; 1.0009x over previous
import functools

import jax
import jax.numpy as jnp
from jax import lax
from jax.experimental import pallas as pl
from jax.experimental.pallas import tpu as pltpu
from jax.experimental.pallas import tpu_sc as plsc

VOCAB = 1000000
EMBED_DIM = 32
HIDDEN_DIM = 64
NUM_TIERS = 4
BATCH = 16384
HIST_LEN = 200

_L = 16


def _make_pool_kernel():
    info = plsc.get_sparse_core_info()
    nc, ns = info.num_cores, info.num_subcores
    nw = nc * ns
    S = BATCH // nw
    CH = 256
    NCH = S // CH
    NBUF = 8

    mesh = plsc.VectorSubcoreMesh(core_axis_name="c", subcore_axis_name="s")

    @functools.partial(
        pl.kernel,
        mesh=mesh,
        out_type=jax.ShapeDtypeStruct((BATCH, EMBED_DIM), jnp.float32),
        scratch_types=[
            pltpu.VMEM((CH, HIST_LEN), jnp.int32),
            pltpu.VMEM((NBUF, HIST_LEN, EMBED_DIM), jnp.float32),
            pltpu.VMEM((CH, EMBED_DIM), jnp.float32),
            pltpu.SemaphoreType.DMA,
            pltpu.SemaphoreType.DMA,
            pltpu.SemaphoreType.DMA,
            pltpu.SemaphoreType.DMA,
            pltpu.SemaphoreType.DMA,
            pltpu.SemaphoreType.DMA,
            pltpu.SemaphoreType.DMA,
            pltpu.SemaphoreType.DMA,
        ],
        compiler_params=pltpu.CompilerParams(use_tc_tiling_on_sc=False),
    )
    def pool(ids_hbm, table_hbm, out_hbm, ids_v, rows_v, sums_v, *sems):
        wid = lax.axis_index("s") * nc + lax.axis_index("c")
        base = wid * S

        def start_gather(i, b):
            pltpu.async_copy(
                table_hbm.at[ids_v.at[i, pl.ds(0, 128)]],
                rows_v.at[b, pl.ds(0, 128)],
                sems[b],
            )
            pltpu.async_copy(
                table_hbm.at[ids_v.at[i, pl.ds(128, HIST_LEN - 128)]],
                rows_v.at[b, pl.ds(128, HIST_LEN - 128)],
                sems[b],
            )

        def wait_gather(b):
            pltpu.make_async_copy(
                table_hbm.at[pl.ds(0, HIST_LEN)], rows_v.at[b], sems[b]
            ).wait()

        def accum_store(i, b):
            def acc_body(j, accs):
                accs = list(accs)
                r = j * 8
                for u in range(8):
                    accs[u % 4] = accs[u % 4] + rows_v[b, r + u, pl.ds(0, _L)]
                    accs[4 + u % 4] = accs[4 + u % 4] + rows_v[b, r + u, pl.ds(_L, _L)]
                return tuple(accs)

            z = jnp.zeros((_L,), jnp.float32)
            accs = lax.fori_loop(0, HIST_LEN // 8, acc_body, (z,) * 8)
            sums_v[i, pl.ds(0, _L)] = (accs[0] + accs[1]) + (accs[2] + accs[3])
            sums_v[i, pl.ds(_L, _L)] = (accs[4] + accs[5]) + (accs[6] + accs[7])

        def chunk_body(c, carry):
            cbase = base + c * CH
            pltpu.sync_copy(ids_hbm.at[pl.ds(cbase, CH)], ids_v)
            for b in range(NBUF - 1):
                start_gather(b, b)

            def group_body(g, carry2):
                for b in range(NBUF):
                    i = g * NBUF + b

                    @pl.when(i + NBUF - 1 < CH)
                    def _():
                        start_gather(i + NBUF - 1, (b + NBUF - 1) % NBUF)

                    wait_gather(b)
                    accum_store(i, b)
                return carry2

            lax.fori_loop(0, CH // NBUF, group_body, 0)
            pltpu.sync_copy(sums_v, out_hbm.at[pl.ds(cbase, CH)])
            return carry

        lax.fori_loop(0, NCH, chunk_body, 0)

    return pool


_pool = _make_pool_kernel()


_QP = 1 << 18


def _lin_body(t0, t1, t2, t3, out_ref):
    out_ref[...] = jnp.concatenate(
        [t0[...].T, t1[...].T, t2[...].T, t3[...].T], axis=1
    )


def _linearize(table_t):
    BM = 8192
    G = _QP // BM
    last_blk = VOCAB // BM
    return pl.pallas_call(
        _lin_body,
        grid=(G,),
        in_specs=[
            pl.BlockSpec(
                (EMBED_DIM, BM),
                lambda g, k=k: (0, jnp.minimum(k * G + g, last_blk)),
            )
            for k in range(4)
        ],
        out_specs=pl.BlockSpec((BM, 128), lambda g: (g, 0)),
        out_shape=jax.ShapeDtypeStruct((_QP, 128), jnp.float32),
    )(table_t, table_t, table_t, table_t)


def _remap_body(ids_ref, out_ref):
    v = ids_ref[...]
    out_ref[...] = ((v & (_QP - 1)) << 2) | (v >> 18)


def _remap(ids):
    BT = 1024
    return pl.pallas_call(
        _remap_body,
        grid=(BATCH // BT,),
        in_specs=[pl.BlockSpec((BT, HIST_LEN), lambda i: (i, 0))],
        out_specs=pl.BlockSpec((BT, HIST_LEN), lambda i: (i, 0)),
        out_shape=jax.ShapeDtypeStruct((BATCH, HIST_LEN), jnp.int32),
    )(ids)


def _mlp_body(ids_ref, sums_ref, w1_ref, b1_ref, w2_ref, b2_ref, out_ref):
    ids = ids_ref[...]
    valid = jnp.sum((ids != 0).astype(jnp.float32), axis=1, keepdims=True)
    rep = sums_ref[...] / jnp.maximum(valid, 1.0)
    x = jnp.dot(rep, w1_ref[...], preferred_element_type=jnp.float32) + b1_ref[...]
    x = jnp.maximum(x, 0.0)
    out_ref[...] = (
        jnp.dot(x, w2_ref[...], preferred_element_type=jnp.float32) + b2_ref[...]
    )


def _mlp(ids, sums, w1t, b1r, w2t, b2r):
    BT = 1024
    return pl.pallas_call(
        _mlp_body,
        grid=(BATCH // BT,),
        in_specs=[
            pl.BlockSpec((BT, HIST_LEN), lambda i: (i, 0)),
            pl.BlockSpec((BT, EMBED_DIM), lambda i: (i, 0)),
            pl.BlockSpec((EMBED_DIM, HIDDEN_DIM), lambda i: (0, 0)),
            pl.BlockSpec((1, HIDDEN_DIM), lambda i: (0, 0)),
            pl.BlockSpec((HIDDEN_DIM, NUM_TIERS), lambda i: (0, 0)),
            pl.BlockSpec((1, NUM_TIERS), lambda i: (0, 0)),
        ],
        out_specs=pl.BlockSpec((BT, NUM_TIERS), lambda i: (i, 0)),
        out_shape=jax.ShapeDtypeStruct((BATCH, NUM_TIERS), jnp.float32),
    )(ids, sums, w1t, b1r, w2t, b2r)


def kernel(input_ids, table, W1, b1, W2, b2):
    lin = _linearize(table.T).reshape(4 * _QP, EMBED_DIM)
    rids = _remap(input_ids)
    sums = _pool(rids, lin)
    return _mlp(
        input_ids,
        sums,
        W1.T,
        b1.reshape(1, HIDDEN_DIM),
        W2.T,
        b2.reshape(1, NUM_TIERS),
    )
